# Initial kernel scaffold; baseline (speedup 1.0000x reference)
import jax
import jax.numpy as jnp
from jax.experimental import pallas as pl


def kernel(X, weight, bias):
    raise NotImplementedError("write your pallas kernel here")


# speedup vs baseline: 1.1811x; 1.1811x over previous
import jax
import jax.numpy as jnp
from jax import lax
from jax.experimental import pallas as pl
from jax.experimental.pallas import tpu as pltpu
from jax.experimental.pallas import tpu_sc as plsc

BATCH = 16384
N_FIELDS = 100
INPUT_DIM = 1000000
NW = 32
ROWS_PER_W = BATCH // NW
IDX_PER_W = ROWS_PER_W * N_FIELDS
MINOR = 128
MAJOR = IDX_PER_W // MINOR
LANES = 16


def _wide_body(x_hbm, tab_hbm, bias_hbm, emb_hbm, out_hbm,
               idx_v, vals_v, sums_v, bias_v, sem):
    c = lax.axis_index("c")
    s = lax.axis_index("s")
    wid = s * 2 + c

    pltpu.sync_copy(x_hbm.at[wid], idx_v)
    pltpu.async_copy(tab_hbm.at[idx_v], vals_v, sem).wait()
    pltpu.sync_copy(vals_v, emb_hbm.at[wid])

    pltpu.sync_copy(bias_hbm, bias_v)
    bias_vec = bias_v[...]
    lane = lax.iota(jnp.int32, LANES)

    def group_body(g, _):
        flat0 = (g * LANES + lane) * N_FIELDS

        def f_body(f, carry):
            acc, flat = carry
            v = plsc.load_gather(vals_v, [flat])
            return acc + v, flat + 1

        acc, _ = lax.fori_loop(
            0, N_FIELDS, f_body,
            (jnp.zeros((LANES,), jnp.float32), flat0))
        sums_v[pl.ds(g * LANES, LANES)] = acc + bias_vec
        return 0

    lax.fori_loop(0, ROWS_PER_W // LANES, group_body, 0)
    pltpu.sync_copy(sums_v, out_hbm.at[wid])


def kernel(X, weight, bias):
    Xr = X.reshape(NW, IDX_PER_W)
    tab = weight.reshape(INPUT_DIM)
    bias16 = jnp.broadcast_to(bias.astype(jnp.float32), (LANES,))
    mesh = plsc.VectorSubcoreMesh(
        core_axis_name="c", subcore_axis_name="s",
        num_cores=2, num_subcores=16)
    emb, out = pl.kernel(
        _wide_body,
        out_type=(
            jax.ShapeDtypeStruct((NW, IDX_PER_W), jnp.float32),
            jax.ShapeDtypeStruct((NW, ROWS_PER_W), jnp.float32),
        ),
        mesh=mesh,
        compiler_params=pltpu.CompilerParams(needs_layout_passes=False),
        scratch_types=[
            pltpu.VMEM((IDX_PER_W,), jnp.int32),
            pltpu.VMEM((IDX_PER_W,), jnp.float32),
            pltpu.VMEM((ROWS_PER_W,), jnp.float32),
            pltpu.VMEM((LANES,), jnp.float32),
            pltpu.SemaphoreType.DMA,
        ],
    )(Xr, tab, bias16)
    return (out.reshape(BATCH, 1), emb.reshape(BATCH, N_FIELDS, 1))


# speedup vs baseline: 2.0206x; 1.7109x over previous
import jax
import jax.numpy as jnp
from jax import lax
from jax.experimental import pallas as pl
from jax.experimental.pallas import tpu as pltpu
from jax.experimental.pallas import tpu_sc as plsc

BATCH = 16384
N_FIELDS = 100
INPUT_DIM = 1000000
NW = 32
BW = BATCH // NW
LANES = 16
GROUPS = BW // LANES


def _wide_body(xt_hbm, tab_hbm, bias_hbm, emb_hbm, out_hbm,
               idx_v, vals_v, sums_v, bias_v, sem, isem):
    c = lax.axis_index("c")
    s = lax.axis_index("s")
    wid = s * 2 + c
    b0 = pl.multiple_of(wid * BW, 8)

    icps = [
        pltpu.async_copy(xt_hbm.at[f, pl.ds(b0, BW)],
                         idx_v.at[pl.ds(f * BW, BW)], isem)
        for f in range(N_FIELDS)
    ]
    pltpu.sync_copy(bias_hbm, bias_v)
    for cp in icps:
        cp.wait()

    tab_row = tab_hbm.at[0]
    cps = [
        pltpu.async_copy(tab_row.at[idx_v.at[pl.ds(f * BW, BW)]],
                         vals_v.at[pl.ds(f * BW, BW)], sem)
        for f in range(N_FIELDS)
    ]
    for cp in cps:
        cp.wait()

    ecps = [
        pltpu.async_copy(vals_v.at[pl.ds(f * BW, BW)],
                         emb_hbm.at[f, pl.ds(b0, BW)], isem)
        for f in range(N_FIELDS)
    ]

    bias_vec = bias_v[...]

    def group_body(g, _):
        col0 = g * LANES
        acc = vals_v[pl.ds(col0, LANES)]
        for f in range(1, N_FIELDS):
            acc = acc + vals_v[pl.ds(f * BW + col0, LANES)]
        sums_v[pl.ds(col0, LANES)] = acc + bias_vec
        return 0

    lax.fori_loop(0, GROUPS, group_body, 0)
    pltpu.sync_copy(sums_v, out_hbm.at[0].at[pl.ds(b0, BW)])
    for cp in ecps:
        cp.wait()


def kernel(X, weight, bias):
    Xt = jnp.transpose(X)
    bias16 = jnp.broadcast_to(bias.astype(jnp.float32), (LANES,))
    mesh = plsc.VectorSubcoreMesh(
        core_axis_name="c", subcore_axis_name="s",
        num_cores=2, num_subcores=16)
    emb_t, out = pl.kernel(
        _wide_body,
        out_type=(
            jax.ShapeDtypeStruct((N_FIELDS, BATCH), jnp.float32),
            jax.ShapeDtypeStruct((1, BATCH), jnp.float32),
        ),
        mesh=mesh,
        compiler_params=pltpu.CompilerParams(needs_layout_passes=False),
        scratch_types=[
            pltpu.VMEM((N_FIELDS * BW,), jnp.int32),
            pltpu.VMEM((N_FIELDS * BW,), jnp.float32),
            pltpu.VMEM((BW,), jnp.float32),
            pltpu.VMEM((LANES,), jnp.float32),
            pltpu.SemaphoreType.DMA,
            pltpu.SemaphoreType.DMA,
        ],
    )(Xt, weight.reshape(1, INPUT_DIM), bias16)
    emb = jnp.transpose(emb_t).reshape(BATCH, N_FIELDS, 1)
    return (out.reshape(BATCH, 1), emb)


# speedup vs baseline: 2.2346x; 1.1059x over previous
import jax
import jax.numpy as jnp
from jax import lax
from jax.experimental import pallas as pl
from jax.experimental.pallas import tpu as pltpu
from jax.experimental.pallas import tpu_sc as plsc

BATCH = 16384
N_FIELDS = 100
INPUT_DIM = 1000000
NW = 32
BW = BATCH // NW
LANES = 16
GROUPS = BW // LANES


def _wide_body(xt_hbm, tab_hbm, bias_hbm, emb_hbm, out_hbm,
               idx_v, vals_v, sums_v, bias_v, sem, isem):
    c = lax.axis_index("c")
    s = lax.axis_index("s")
    wid = s * 2 + c
    b0 = pl.multiple_of(wid * BW, 8)
    tab_row = tab_hbm.at[0]

    def stage_idx(f, _):
        off = pl.multiple_of(f * BW, 8)
        pltpu.async_copy(xt_hbm.at[f, pl.ds(b0, BW)],
                         idx_v.at[pl.ds(off, BW)], isem)
        return 0
    lax.fori_loop(0, N_FIELDS, stage_idx, 0)

    pltpu.sync_copy(bias_hbm, bias_v)
    bias_vec = bias_v[...]

    def init_sums(g, _):
        sums_v[pl.ds(g * LANES, LANES)] = bias_vec
        return 0
    lax.fori_loop(0, GROUPS, init_sums, 0)

    def drain_idx(f, _):
        pltpu.make_async_copy(xt_hbm.at[0, pl.ds(b0, BW)],
                              idx_v.at[pl.ds(0, BW)], isem).wait()
        return 0
    lax.fori_loop(0, N_FIELDS, drain_idx, 0)

    def fire(f, _):
        off = pl.multiple_of(f * BW, 8)
        pltpu.async_copy(tab_row.at[idx_v.at[pl.ds(off, BW)]],
                         vals_v.at[pl.ds(off, BW)], sem)
        return 0
    lax.fori_loop(0, N_FIELDS, fire, 0)

    def drain(f, _):
        off = pl.multiple_of(f * BW, 8)
        pltpu.make_async_copy(tab_row.at[idx_v.at[pl.ds(off, BW)]],
                              vals_v.at[pl.ds(off, BW)], sem).wait()
        pltpu.async_copy(vals_v.at[pl.ds(off, BW)],
                         emb_hbm.at[f, 0, pl.ds(b0, BW)], isem)

        def acc(g, _):
            o = g * LANES
            sums_v[pl.ds(o, LANES)] = (
                sums_v[pl.ds(o, LANES)] + vals_v[pl.ds(off + o, LANES)])
            return 0
        lax.fori_loop(0, GROUPS, acc, 0)
        return 0
    lax.fori_loop(0, N_FIELDS, drain, 0)

    pltpu.sync_copy(sums_v, out_hbm.at[0].at[pl.ds(b0, BW)])

    def drain_emb(f, _):
        pltpu.make_async_copy(vals_v.at[pl.ds(0, BW)],
                              emb_hbm.at[0, 0, pl.ds(b0, BW)], isem).wait()
        return 0
    lax.fori_loop(0, N_FIELDS, drain_emb, 0)


def kernel(X, weight, bias):
    Xt = jnp.transpose(X)
    bias16 = jnp.broadcast_to(bias.astype(jnp.float32), (LANES,))
    mesh = plsc.VectorSubcoreMesh(
        core_axis_name="c", subcore_axis_name="s",
        num_cores=2, num_subcores=16)
    emb_t, out = pl.kernel(
        _wide_body,
        out_type=(
            jax.ShapeDtypeStruct((N_FIELDS, 1, BATCH), jnp.float32),
            jax.ShapeDtypeStruct((1, BATCH), jnp.float32),
        ),
        mesh=mesh,
        compiler_params=pltpu.CompilerParams(needs_layout_passes=False),
        scratch_types=[
            pltpu.VMEM((N_FIELDS * BW,), jnp.int32),
            pltpu.VMEM((N_FIELDS * BW,), jnp.float32),
            pltpu.VMEM((BW,), jnp.float32),
            pltpu.VMEM((LANES,), jnp.float32),
            pltpu.SemaphoreType.DMA,
            pltpu.SemaphoreType.DMA,
        ],
    )(Xt, weight.reshape(1, INPUT_DIM), bias16)
    emb = jnp.transpose(emb_t, (2, 0, 1))
    return (out.reshape(BATCH, 1), emb)


# speedup vs baseline: 2.2538x; 1.0086x over previous
import jax
import jax.numpy as jnp
from jax import lax
from jax.experimental import pallas as pl
from jax.experimental.pallas import tpu as pltpu
from jax.experimental.pallas import tpu_sc as plsc

BATCH = 16384
N_FIELDS = 100
INPUT_DIM = 1000000
NW = 32
BW = BATCH // NW
LANES = 16
GROUPS = BW // LANES


def _wide_body(xt_hbm, tab_hbm, bias_hbm, emb_hbm, out_hbm,
               idx_v, vals_v, sums_v, bias_v, sem, isem):
    c = lax.axis_index("c")
    s = lax.axis_index("s")
    wid = s * 2 + c
    b0 = pl.multiple_of(wid * BW, 8)

    icps = [
        pltpu.async_copy(xt_hbm.at[f, pl.ds(b0, BW)],
                         idx_v.at[pl.ds(f * BW, BW)], isem)
        for f in range(N_FIELDS)
    ]
    pltpu.sync_copy(bias_hbm, bias_v)
    for cp in icps:
        cp.wait()

    tab_row = tab_hbm.at[0]
    cps = [
        pltpu.async_copy(tab_row.at[idx_v.at[pl.ds(f * BW, BW)]],
                         vals_v.at[pl.ds(f * BW, BW)], sem)
        for f in range(N_FIELDS)
    ]
    for cp in cps:
        cp.wait()

    ecps = [
        pltpu.async_copy(vals_v.at[pl.ds(f * BW, BW)],
                         emb_hbm.at[f, 0, pl.ds(b0, BW)], isem)
        for f in range(N_FIELDS)
    ]

    bias_vec = bias_v[...]

    def group_body(g, _):
        col0 = g * LANES
        acc = vals_v[pl.ds(col0, LANES)]
        for f in range(1, N_FIELDS):
            acc = acc + vals_v[pl.ds(f * BW + col0, LANES)]
        sums_v[pl.ds(col0, LANES)] = acc + bias_vec
        return 0

    lax.fori_loop(0, GROUPS, group_body, 0)
    pltpu.sync_copy(sums_v, out_hbm.at[0].at[pl.ds(b0, BW)])
    for cp in ecps:
        cp.wait()


def kernel(X, weight, bias):
    Xt = jnp.transpose(X)
    bias16 = jnp.broadcast_to(bias.astype(jnp.float32), (LANES,))
    mesh = plsc.VectorSubcoreMesh(
        core_axis_name="c", subcore_axis_name="s",
        num_cores=2, num_subcores=16)
    emb_t, out = pl.kernel(
        _wide_body,
        out_type=(
            jax.ShapeDtypeStruct((N_FIELDS, 1, BATCH), jnp.float32),
            jax.ShapeDtypeStruct((1, BATCH), jnp.float32),
        ),
        mesh=mesh,
        compiler_params=pltpu.CompilerParams(needs_layout_passes=False),
        scratch_types=[
            pltpu.VMEM((N_FIELDS * BW,), jnp.int32),
            pltpu.VMEM((N_FIELDS * BW,), jnp.float32),
            pltpu.VMEM((BW,), jnp.float32),
            pltpu.VMEM((LANES,), jnp.float32),
            pltpu.SemaphoreType.DMA,
            pltpu.SemaphoreType.DMA,
        ],
    )(Xt, weight.reshape(1, INPUT_DIM), bias16)
    emb = jnp.transpose(emb_t, (2, 0, 1))
    return (out.reshape(BATCH, 1), emb)


# speedup vs baseline: 2.3066x; 1.0235x over previous
import jax
import jax.numpy as jnp
from jax import lax
from jax.experimental import pallas as pl
from jax.experimental.pallas import tpu as pltpu
from jax.experimental.pallas import tpu_sc as plsc

BATCH = 16384
N_FIELDS = 100
INPUT_DIM = 1000000
NW = 32
BW = BATCH // NW
LANES = 16
GROUPS = BW // LANES


def _wide_body(xt_hbm, tab_hbm, bias_hbm, emb_hbm, out_hbm,
               idx_v, vals_v, sums_v, bias_v, sem, isem):
    c = lax.axis_index("c")
    s = lax.axis_index("s")
    wid = s * 2 + c
    b0 = pl.multiple_of(wid * BW, 8)

    icps = [
        pltpu.async_copy(xt_hbm.at[f, pl.ds(b0, BW)],
                         idx_v.at[pl.ds(f * BW, BW)], isem)
        for f in range(N_FIELDS)
    ]
    pltpu.sync_copy(bias_hbm, bias_v)
    for cp in icps:
        cp.wait()

    tab_row = tab_hbm.at[0]
    cps = [
        pltpu.async_copy(tab_row.at[idx_v.at[pl.ds(f * BW, BW)]],
                         vals_v.at[pl.ds(f * BW, BW)], sem)
        for f in range(N_FIELDS)
    ]
    for cp in cps:
        cp.wait()

    ecps = [
        pltpu.async_copy(vals_v.at[pl.ds(f * BW, BW)],
                         emb_hbm.at[f, 0, pl.ds(b0, BW)], isem)
        for f in range(N_FIELDS)
    ]

    bias_vec = bias_v[...]

    def group_body(g, _):
        col0 = g * LANES
        accs = [vals_v[pl.ds(a * BW + col0, LANES)] for a in range(4)]
        for f in range(4, N_FIELDS):
            accs[f % 4] = accs[f % 4] + vals_v[pl.ds(f * BW + col0, LANES)]
        sums_v[pl.ds(col0, LANES)] = (
            (accs[0] + accs[1]) + (accs[2] + accs[3]) + bias_vec)
        return 0

    lax.fori_loop(0, GROUPS, group_body, 0)
    pltpu.sync_copy(sums_v, out_hbm.at[0].at[pl.ds(b0, BW)])
    for cp in ecps:
        cp.wait()


def kernel(X, weight, bias):
    Xt = jnp.transpose(X)
    bias16 = jnp.broadcast_to(bias.astype(jnp.float32), (LANES,))
    mesh = plsc.VectorSubcoreMesh(
        core_axis_name="c", subcore_axis_name="s",
        num_cores=2, num_subcores=16)
    emb_t, out = pl.kernel(
        _wide_body,
        out_type=(
            jax.ShapeDtypeStruct((N_FIELDS, 1, BATCH), jnp.float32),
            jax.ShapeDtypeStruct((1, BATCH), jnp.float32),
        ),
        mesh=mesh,
        compiler_params=pltpu.CompilerParams(needs_layout_passes=False),
        scratch_types=[
            pltpu.VMEM((N_FIELDS * BW,), jnp.int32),
            pltpu.VMEM((N_FIELDS * BW,), jnp.float32),
            pltpu.VMEM((BW,), jnp.float32),
            pltpu.VMEM((LANES,), jnp.float32),
            pltpu.SemaphoreType.DMA,
            pltpu.SemaphoreType.DMA,
        ],
    )(Xt, weight.reshape(1, INPUT_DIM), bias16)
    emb = jnp.transpose(emb_t, (2, 0, 1))
    return (out.reshape(BATCH, 1), emb)
